# baseline (speedup 1.0000x reference)
import functools

import jax
import jax.numpy as jnp
from jax.experimental import pallas as pl

N, DIM, NODE_IN, E, E3 = 50000, 64, 8, 800000, 400000
D2, D6 = DIM * 2, DIM * 6
RRELU_SLOPE = (1.0 / 8.0 + 1.0 / 3.0) / 2.0
NBLK = 2000
EBLK = 4000


def _rr(y):
    return jnp.where(y >= 0, y, RRELU_SLOPE * y)


def _stats_body(x_ref, o_ref):
    i = pl.program_id(0)

    @pl.when(i == 0)
    def _():
        o_ref[...] = jnp.zeros_like(o_ref)

    xb = x_ref[...]
    s = jnp.sum(xb, axis=0, keepdims=True)
    s2 = jnp.sum(xb * xb, axis=0, keepdims=True)
    o_ref[...] += jnp.concatenate([s, s2], axis=0)


def _col_stats(x, blk):
    n, c = x.shape
    return pl.pallas_call(
        _stats_body,
        grid=(n // blk,),
        in_specs=[pl.BlockSpec((blk, c), lambda i: (i, 0))],
        out_specs=pl.BlockSpec((2, c), lambda i: (0, 0)),
        out_shape=jax.ShapeDtypeStruct((2, c), jnp.float32),
    )(x)


def _encode_body(st_ref, x_ref, g_ref, b_ref, w_ref, bn_ref, o_ref):
    n = jnp.float32(N)
    mu = st_ref[0:1, :] / n
    var = st_ref[1:2, :] / n - mu * mu
    a = g_ref[...] / jnp.sqrt(var + 1e-5)
    bb = b_ref[...] - mu * a
    xb = x_ref[...] * a + bb
    y = jnp.dot(xb, w_ref[...], preferred_element_type=jnp.float32) + bn_ref[...]
    o_ref[...] = _rr(y)


def _encode(x, gx, bx, WnT, bn_):
    st = _col_stats(x, NBLK)
    return pl.pallas_call(
        _encode_body,
        grid=(N // NBLK,),
        in_specs=[
            pl.BlockSpec((2, NODE_IN), lambda i: (0, 0)),
            pl.BlockSpec((NBLK, NODE_IN), lambda i: (i, 0)),
            pl.BlockSpec((1, NODE_IN), lambda i: (0, 0)),
            pl.BlockSpec((1, NODE_IN), lambda i: (0, 0)),
            pl.BlockSpec((NODE_IN, DIM), lambda i: (0, 0)),
            pl.BlockSpec((1, DIM), lambda i: (0, 0)),
        ],
        out_specs=pl.BlockSpec((NBLK, DIM), lambda i: (i, 0)),
        out_shape=jax.ShapeDtypeStruct((N, DIM), jnp.float32),
    )(st, x, gx.reshape(1, -1), bx.reshape(1, -1), WnT, bn_.reshape(1, -1))


def _proj_body(x_ref, w_ref, att_ref, xp_ref, a2_ref):
    xp = jnp.dot(x_ref[...], w_ref[...], preferred_element_type=jnp.float32)
    xp_ref[...] = xp
    a2_ref[...] = jnp.dot(xp, att_ref[...], preferred_element_type=jnp.float32)


def _project(out, WT, att2):
    c = WT.shape[0]
    return pl.pallas_call(
        _proj_body,
        grid=(N // NBLK,),
        in_specs=[
            pl.BlockSpec((NBLK, c), lambda i: (i, 0)),
            pl.BlockSpec((c, c), lambda i: (0, 0)),
            pl.BlockSpec((c, 2), lambda i: (0, 0)),
        ],
        out_specs=(
            pl.BlockSpec((NBLK, c), lambda i: (i, 0)),
            pl.BlockSpec((NBLK, 2), lambda i: (i, 0)),
        ),
        out_shape=(
            jax.ShapeDtypeStruct((N, c), jnp.float32),
            jax.ShapeDtypeStruct((N, 2), jnp.float32),
        ),
    )(out, WT, att2)


def _gru_body(num_ref, den_ref, b_ref, h_ref,
              wir_ref, wiz_ref, win_ref, whr_ref, whz_ref, whn_ref,
              bi_ref, bh_ref, o_ref):
    c = num_ref.shape[1]
    m = _rr(num_ref[...] / den_ref[...] + b_ref[...])
    h = h_ref[...]
    dot = lambda a, w: jnp.dot(a, w, preferred_element_type=jnp.float32)
    bi = bi_ref[...]
    bh = bh_ref[...]
    r = jax.nn.sigmoid(dot(m, wir_ref[...]) + bi[0:1, :] + dot(h, whr_ref[...]) + bh[0:1, :])
    z = jax.nn.sigmoid(dot(m, wiz_ref[...]) + bi[1:2, :] + dot(h, whz_ref[...]) + bh[1:2, :])
    nn = jnp.tanh(dot(m, win_ref[...]) + bi[2:3, :] + r * (dot(h, whn_ref[...]) + bh[2:3, :]))
    o_ref[...] = (1.0 - z) * nn + z * h


def _gru(num, den, b, h, WihT, WhhT, bih, bhh):
    c = num.shape[1]
    wir, wiz, win = WihT[:, :c], WihT[:, c:2 * c], WihT[:, 2 * c:]
    whr, whz, whn = WhhT[:, :c], WhhT[:, c:2 * c], WhhT[:, 2 * c:]
    bi = bih.reshape(3, c)
    bh = bhh.reshape(3, c)
    full = lambda r, cc: pl.BlockSpec((r, cc), lambda i: (0, 0))
    row = lambda cc: pl.BlockSpec((NBLK, cc), lambda i: (i, 0))
    return pl.pallas_call(
        _gru_body,
        grid=(N // NBLK,),
        in_specs=[row(c), row(1), full(1, c), row(c),
                  full(c, c), full(c, c), full(c, c), full(c, c), full(c, c), full(c, c),
                  full(3, c), full(3, c)],
        out_specs=row(c),
        out_shape=jax.ShapeDtypeStruct((N, c), jnp.float32),
    )(num, den, b.reshape(1, c), h, wir, wiz, win, whr, whz, whn, bi, bh)


def _cmlp_body(st_ref, x_ref, g_ref, b_ref, w1_ref, b1_ref, w2_ref, b2_ref, o_ref):
    n = jnp.float32(N)
    mu = st_ref[0:1, :] / n
    var = st_ref[1:2, :] / n - mu * mu
    a = g_ref[...] / jnp.sqrt(var + 1e-5)
    bb = b_ref[...] - mu * a
    xb = x_ref[...] * a + bb
    c1 = _rr(jnp.dot(xb, w1_ref[...], preferred_element_type=jnp.float32) + b1_ref[...])
    o_ref[...] = _rr(jnp.dot(c1, w2_ref[...], preferred_element_type=jnp.float32) + b2_ref[...])


def _cmlp(out, gc, bc_, Wc1T, bc1, Wc2T, bc2):
    st = _col_stats(out, NBLK)
    return pl.pallas_call(
        _cmlp_body,
        grid=(N // NBLK,),
        in_specs=[
            pl.BlockSpec((2, DIM), lambda i: (0, 0)),
            pl.BlockSpec((NBLK, DIM), lambda i: (i, 0)),
            pl.BlockSpec((1, DIM), lambda i: (0, 0)),
            pl.BlockSpec((1, DIM), lambda i: (0, 0)),
            pl.BlockSpec((DIM, D2), lambda i: (0, 0)),
            pl.BlockSpec((1, D2), lambda i: (0, 0)),
            pl.BlockSpec((D2, D2), lambda i: (0, 0)),
            pl.BlockSpec((1, D2), lambda i: (0, 0)),
        ],
        out_specs=pl.BlockSpec((NBLK, D2), lambda i: (i, 0)),
        out_shape=jax.ShapeDtypeStruct((N, D2), jnp.float32),
    )(st, out, gc.reshape(1, -1), bc_.reshape(1, -1), Wc1T, bc1.reshape(1, -1),
      Wc2T, bc2.reshape(1, -1))


def _yhat(t0, t1):
    return jnp.concatenate([(t0 + t1) * 0.5, t0 * t1, (t0 - t1) ** 2], axis=1)


def _ystats_body(t0_ref, t1_ref, o_ref):
    i = pl.program_id(0)

    @pl.when(i == 0)
    def _():
        o_ref[...] = jnp.zeros_like(o_ref)

    y = _yhat(t0_ref[...], t1_ref[...])
    s = jnp.sum(y, axis=0, keepdims=True)
    s2 = jnp.sum(y * y, axis=0, keepdims=True)
    o_ref[...] += jnp.concatenate([s, s2], axis=0)


def _readout_body(st_ref, t0_ref, t1_ref, ea_ref, g_ref, b_ref, wl_ref, wb_ref, o_ref):
    n = jnp.float32(E3)
    mu = st_ref[0:1, :] / n
    var = st_ref[1:2, :] / n - mu * mu
    a = g_ref[...] / jnp.sqrt(var + 1e-5)
    bb = b_ref[...] - mu * a
    y = _yhat(t0_ref[...], t1_ref[...]) * a + bb
    ea = ea_ref[...]
    w = jnp.dot(ea, wl_ref[...], preferred_element_type=jnp.float32)
    bvec = jnp.dot(ea, wb_ref[...], preferred_element_type=jnp.float32)
    o_ref[...] = jnp.sum(y * w, axis=1, keepdims=True) + bvec


def _readout(t0, t1, ea, gn, bn2, WlwT, WlbT):
    st = pl.pallas_call(
        _ystats_body,
        grid=(E3 // EBLK,),
        in_specs=[pl.BlockSpec((EBLK, D2), lambda i: (i, 0)),
                  pl.BlockSpec((EBLK, D2), lambda i: (i, 0))],
        out_specs=pl.BlockSpec((2, D6), lambda i: (0, 0)),
        out_shape=jax.ShapeDtypeStruct((2, D6), jnp.float32),
    )(t0, t1)
    res = pl.pallas_call(
        _readout_body,
        grid=(E3 // EBLK,),
        in_specs=[
            pl.BlockSpec((2, D6), lambda i: (0, 0)),
            pl.BlockSpec((EBLK, D2), lambda i: (i, 0)),
            pl.BlockSpec((EBLK, D2), lambda i: (i, 0)),
            pl.BlockSpec((EBLK, 8), lambda i: (i, 0)),
            pl.BlockSpec((1, D6), lambda i: (0, 0)),
            pl.BlockSpec((1, D6), lambda i: (0, 0)),
            pl.BlockSpec((8, D6), lambda i: (0, 0)),
            pl.BlockSpec((8, 1), lambda i: (0, 0)),
        ],
        out_specs=pl.BlockSpec((EBLK, 1), lambda i: (i, 0)),
        out_shape=jax.ShapeDtypeStruct((E3, 1), jnp.float32),
    )(st, t0, t1, ea, gn.reshape(1, -1), bn2.reshape(1, -1), WlwT, WlbT)
    return res[:, 0]


def _gat_edges(xp, a2, src, dst):
    alpha = a2[src, 0] + a2[dst, 1]
    alpha = jnp.where(alpha >= 0, alpha, 0.2 * alpha)
    amax = jax.ops.segment_max(alpha, dst, num_segments=N)
    ex = jnp.exp(alpha - amax[dst])
    den = jax.ops.segment_sum(ex, dst, num_segments=N)
    num = jax.ops.segment_sum(ex[:, None] * xp[src], dst, num_segments=N)
    return num, den[:, None]


def kernel(x, edge_index, edge_index3, edge_attr3, gx, bx, Wn, bn_, W1, as1, ad1,
           b1, Wih1, Whh1, bih1, bhh1, gc, bc_, Wc1, bc1, Wc2, bc2, W2, as2, ad2,
           b2, Wih2, Whh2, bih2, bhh2, gn, bn2, Wlw, Wlb):
    sl = jnp.arange(N, dtype=edge_index.dtype)
    src1 = jnp.concatenate([edge_index[0], sl])
    dst1 = jnp.concatenate([edge_index[1], sl])
    ei3 = jnp.concatenate([edge_index3, edge_index3[::-1]], axis=1)
    src2 = jnp.concatenate([ei3[0], sl])
    dst2 = jnp.concatenate([ei3[1], sl])

    out = _encode(x, gx, bx, Wn.T, bn_)
    att2_1 = jnp.stack([as1, ad1], axis=1)
    h = out
    for _ in range(2):
        xp, a2 = _project(out, W1.T, att2_1)
        num, den = _gat_edges(xp, a2, src1, dst1)
        h = _gru(num, den, b1, h, Wih1.T, Whh1.T, bih1, bhh1)
        out = h

    out = _cmlp(out, gc, bc_, Wc1.T, bc1, Wc2.T, bc2)
    att2_2 = jnp.stack([as2, ad2], axis=1)
    h = out
    for _ in range(2):
        xp, a2 = _project(out, W2.T, att2_2)
        num, den = _gat_edges(xp, a2, src2, dst2)
        h = _gru(num, den, b2, h, Wih2.T, Whh2.T, bih2, bhh2)
        out = h

    t0 = out[edge_index3[0]]
    t1 = out[edge_index3[1]]
    return _readout(t0, t1, edge_attr3, gn, bn2, Wlw.T, Wlb.T)
